# baseline (speedup 1.0000x reference)
import functools

import jax
import jax.numpy as jnp
from jax import lax
from jax.experimental import pallas as pl
from jax.experimental.pallas import tpu as pltpu
from jax.experimental.pallas import tpu_sc as plsc

NUM_T = 1000
IMG_SHAPE = (3, 64, 64)
BATCH = 1024
FEAT = 3 * 64 * 64

_NC = 2
_NS = 16
_L = 16
_NW = _NC * _NS
_PER_W = BATCH // _NW
_TBL_PAD = 1024


def _schedule_tables():
    scale = 1000.0 / NUM_T
    beta = jnp.linspace(scale * 0.0001, scale * 0.02, NUM_T, dtype=jnp.float32)
    alpha_cum = jnp.cumprod(1.0 - beta, axis=0)
    sqrt_ac = jnp.sqrt(alpha_cum)
    sqrt_omac = jnp.sqrt(1.0 - alpha_cum)
    pad = _TBL_PAD - NUM_T
    return (jnp.pad(sqrt_ac, (0, pad)), jnp.pad(sqrt_omac, (0, pad)))


def _sc_gather_body(ts_hbm, ac_hbm, om_hbm, coef_hbm, std_hbm,
                    ac_v, om_v, idx_v, coef_v, std_v):
    wid = lax.axis_index("s") * _NC + lax.axis_index("c")
    base = wid * _PER_W
    pltpu.sync_copy(ac_hbm, ac_v)
    pltpu.sync_copy(om_hbm, om_v)
    pltpu.sync_copy(ts_hbm.at[pl.ds(base, _PER_W)], idx_v)
    for j in range(_PER_W // _L):
        idx = idx_v[pl.ds(j * _L, _L)]
        coef_v[pl.ds(j * _L, _L)] = plsc.load_gather(ac_v, [idx])
        std_v[pl.ds(j * _L, _L)] = plsc.load_gather(om_v, [idx])
    pltpu.sync_copy(coef_v, coef_hbm.at[pl.ds(base, _PER_W)])
    pltpu.sync_copy(std_v, std_hbm.at[pl.ds(base, _PER_W)])


_sc_gather = pl.kernel(
    _sc_gather_body,
    out_type=(
        jax.ShapeDtypeStruct((BATCH,), jnp.float32),
        jax.ShapeDtypeStruct((BATCH,), jnp.float32),
    ),
    mesh=plsc.VectorSubcoreMesh(core_axis_name="c", subcore_axis_name="s"),
    compiler_params=pltpu.CompilerParams(needs_layout_passes=False),
    scratch_types=[
        pltpu.VMEM((_TBL_PAD,), jnp.float32),
        pltpu.VMEM((_TBL_PAD,), jnp.float32),
        pltpu.VMEM((_PER_W,), jnp.int32),
        pltpu.VMEM((_PER_W,), jnp.float32),
        pltpu.VMEM((_PER_W,), jnp.float32),
    ],
)


_ROWS = 32


def _scale_body(coef_ref, std_ref, x0_ref, eps_ref, out_ref):
    out_ref[...] = coef_ref[...] * x0_ref[...] + std_ref[...] * eps_ref[...]


def _tc_scale(coef, std, x0, eps):
    grid = (BATCH // _ROWS,)
    img_spec = pl.BlockSpec((_ROWS,) + IMG_SHAPE, lambda i: (i, 0, 0, 0))
    s_spec = pl.BlockSpec((_ROWS, 1, 1, 1), lambda i: (i, 0, 0, 0))
    return pl.pallas_call(
        _scale_body,
        grid=grid,
        in_specs=[s_spec, s_spec, img_spec, img_spec],
        out_specs=img_spec,
        out_shape=jax.ShapeDtypeStruct((BATCH,) + IMG_SHAPE, jnp.float32),
    )(coef, std, x0, eps)


def kernel(x0, timesteps, eps):
    sqrt_ac, sqrt_omac = _schedule_tables()
    coef, std = _sc_gather(timesteps.astype(jnp.int32), sqrt_ac, sqrt_omac)
    sample = _tc_scale(coef.reshape(BATCH, 1, 1, 1),
                       std.reshape(BATCH, 1, 1, 1), x0, eps)
    return (sample, eps)


# speedup vs baseline: 1.5160x; 1.5160x over previous
import functools

import jax
import jax.numpy as jnp
from jax import lax
from jax.experimental import pallas as pl
from jax.experimental.pallas import tpu as pltpu
from jax.experimental.pallas import tpu_sc as plsc

NUM_T = 1000
IMG_SHAPE = (3, 64, 64)
BATCH = 1024
FEAT = 3 * 64 * 64

_NC = 2
_NS = 16
_L = 16
_NW = _NC * _NS
_PER_W = BATCH // _NW
_TBL_PAD = 1024


def _schedule_tables():
    scale = 1000.0 / NUM_T
    beta = jnp.linspace(scale * 0.0001, scale * 0.02, NUM_T, dtype=jnp.float32)
    alpha_cum = jnp.cumprod(1.0 - beta, axis=0)
    sqrt_ac = jnp.sqrt(alpha_cum)
    sqrt_omac = jnp.sqrt(1.0 - alpha_cum)
    pad = _TBL_PAD - NUM_T
    return (jnp.pad(sqrt_ac, (0, pad)), jnp.pad(sqrt_omac, (0, pad)))


def _sc_gather_body(ts_hbm, ac_hbm, om_hbm, coef_hbm, std_hbm,
                    ac_v, om_v, idx_v, coef_v, std_v):
    wid = lax.axis_index("s") * _NC + lax.axis_index("c")
    base = wid * _PER_W
    pltpu.sync_copy(ac_hbm, ac_v)
    pltpu.sync_copy(om_hbm, om_v)
    pltpu.sync_copy(ts_hbm.at[pl.ds(base, _PER_W)], idx_v)
    for j in range(_PER_W // _L):
        idx = idx_v[pl.ds(j * _L, _L)]
        coef_v[pl.ds(j * _L, _L)] = plsc.load_gather(ac_v, [idx])
        std_v[pl.ds(j * _L, _L)] = plsc.load_gather(om_v, [idx])
    pltpu.sync_copy(coef_v, coef_hbm.at[pl.ds(base, _PER_W)])
    pltpu.sync_copy(std_v, std_hbm.at[pl.ds(base, _PER_W)])


_sc_gather = pl.kernel(
    _sc_gather_body,
    out_type=(
        jax.ShapeDtypeStruct((BATCH,), jnp.float32),
        jax.ShapeDtypeStruct((BATCH,), jnp.float32),
    ),
    mesh=plsc.VectorSubcoreMesh(core_axis_name="c", subcore_axis_name="s"),
    compiler_params=pltpu.CompilerParams(needs_layout_passes=False),
    scratch_types=[
        pltpu.VMEM((_TBL_PAD,), jnp.float32),
        pltpu.VMEM((_TBL_PAD,), jnp.float32),
        pltpu.VMEM((_PER_W,), jnp.int32),
        pltpu.VMEM((_PER_W,), jnp.float32),
        pltpu.VMEM((_PER_W,), jnp.float32),
    ],
)


_ROWS = 16


def _scale_body(coef_ref, std_ref, x0_ref, eps_ref, out_ref):
    out_ref[...] = coef_ref[...] * x0_ref[...] + std_ref[...] * eps_ref[...]


def _tc_scale(coef, std, x2, e2):
    grid = (BATCH // _ROWS,)
    row_spec = pl.BlockSpec((_ROWS, FEAT), lambda i: (i, 0))
    s_spec = pl.BlockSpec((_ROWS, 1), lambda i: (i, 0))
    return pl.pallas_call(
        _scale_body,
        grid=grid,
        in_specs=[s_spec, s_spec, row_spec, row_spec],
        out_specs=row_spec,
        out_shape=jax.ShapeDtypeStruct((BATCH, FEAT), jnp.float32),
    )(coef, std, x2, e2)


def kernel(x0, timesteps, eps):
    sqrt_ac, sqrt_omac = _schedule_tables()
    coef, std = _sc_gather(timesteps.astype(jnp.int32), sqrt_ac, sqrt_omac)
    sample = _tc_scale(coef.reshape(BATCH, 1), std.reshape(BATCH, 1),
                       x0.reshape(BATCH, FEAT), eps.reshape(BATCH, FEAT))
    return (sample.reshape(x0.shape), eps)


# speedup vs baseline: 1.6618x; 1.0962x over previous
import functools

import jax
import jax.numpy as jnp
from jax import lax
from jax.experimental import pallas as pl
from jax.experimental.pallas import tpu as pltpu
from jax.experimental.pallas import tpu_sc as plsc

NUM_T = 1000
IMG_SHAPE = (3, 64, 64)
BATCH = 1024
FEAT = 3 * 64 * 64

_NC = 2
_NS = 16
_L = 16
_NW = _NC * _NS
_PER_W = BATCH // _NW
_TBL_PAD = 1024


def _schedule_tables():
    scale = 1000.0 / NUM_T
    beta = jnp.linspace(scale * 0.0001, scale * 0.02, NUM_T, dtype=jnp.float32)
    alpha_cum = jnp.cumprod(1.0 - beta, axis=0)
    sqrt_ac = jnp.sqrt(alpha_cum)
    sqrt_omac = jnp.sqrt(1.0 - alpha_cum)
    pad = _TBL_PAD - NUM_T
    return (jnp.pad(sqrt_ac, (0, pad)), jnp.pad(sqrt_omac, (0, pad)))


def _sc_gather_body(ts_hbm, ac_hbm, om_hbm, coef_hbm, std_hbm,
                    ac_v, om_v, idx_v, coef_v, std_v):
    wid = lax.axis_index("s") * _NC + lax.axis_index("c")
    base = wid * _PER_W
    pltpu.sync_copy(ac_hbm, ac_v)
    pltpu.sync_copy(om_hbm, om_v)
    pltpu.sync_copy(ts_hbm.at[pl.ds(base, _PER_W)], idx_v)
    for j in range(_PER_W // _L):
        idx = idx_v[pl.ds(j * _L, _L)]
        coef_v[pl.ds(j * _L, _L)] = plsc.load_gather(ac_v, [idx])
        std_v[pl.ds(j * _L, _L)] = plsc.load_gather(om_v, [idx])
    pltpu.sync_copy(coef_v, coef_hbm.at[pl.ds(base, _PER_W)])
    pltpu.sync_copy(std_v, std_hbm.at[pl.ds(base, _PER_W)])


@functools.lru_cache(maxsize=None)
def _sc_gather_fn():
    return pl.kernel(
        _sc_gather_body,
        out_type=(
            jax.ShapeDtypeStruct((BATCH,), jnp.float32),
            jax.ShapeDtypeStruct((BATCH,), jnp.float32),
        ),
        mesh=plsc.VectorSubcoreMesh(core_axis_name="c", subcore_axis_name="s"),
        compiler_params=pltpu.CompilerParams(needs_layout_passes=False),
        scratch_types=[
            pltpu.VMEM((_TBL_PAD,), jnp.float32),
            pltpu.VMEM((_TBL_PAD,), jnp.float32),
            pltpu.VMEM((_PER_W,), jnp.int32),
            pltpu.VMEM((_PER_W,), jnp.float32),
            pltpu.VMEM((_PER_W,), jnp.float32),
        ],
    )


_R = 32
_NCHUNK = BATCH // _R
_NBUF = 4


def _scale_body(coef_hbm, std_hbm, x_hbm, e_hbm, out_hbm,
                coef_v, std_v, xb, eb, ob, sem_c, sem_in, sem_out):
    cp_c = pltpu.make_async_copy(coef_hbm, coef_v, sem_c.at[0])
    cp_s = pltpu.make_async_copy(std_hbm, std_v, sem_c.at[1])
    cp_c.start()
    cp_s.start()

    def start_in(i, b):
        pltpu.make_async_copy(x_hbm.at[pl.ds(i * _R, _R), :], xb.at[b],
                              sem_in.at[b, 0]).start()
        pltpu.make_async_copy(e_hbm.at[pl.ds(i * _R, _R), :], eb.at[b],
                              sem_in.at[b, 1]).start()

    for b in range(_NBUF):
        start_in(b, b)
    cp_c.wait()
    cp_s.wait()

    def step(i, _):
        b = lax.rem(i, _NBUF)
        pltpu.make_async_copy(x_hbm.at[pl.ds(0, _R), :], xb.at[b],
                              sem_in.at[b, 0]).wait()
        pltpu.make_async_copy(e_hbm.at[pl.ds(0, _R), :], eb.at[b],
                              sem_in.at[b, 1]).wait()

        @pl.when(i >= _NBUF)
        def _():
            pltpu.make_async_copy(ob.at[b], out_hbm.at[pl.ds(0, _R), :],
                                  sem_out.at[b]).wait()

        coef = coef_v[pl.ds(i * _R, _R), :]
        std = std_v[pl.ds(i * _R, _R), :]
        ob[b] = coef * xb[b] + std * eb[b]
        pltpu.make_async_copy(ob.at[b], out_hbm.at[pl.ds(i * _R, _R), :],
                              sem_out.at[b]).start()

        @pl.when(i + _NBUF < _NCHUNK)
        def _():
            start_in(i + _NBUF, b)

        return 0

    lax.fori_loop(0, _NCHUNK, step, 0)
    for b in range(_NBUF):
        pltpu.make_async_copy(ob.at[b], out_hbm.at[pl.ds(0, _R), :],
                              sem_out.at[b]).wait()


def _tc_scale(coef, std, x2, e2):
    return pl.pallas_call(
        _scale_body,
        in_specs=[pl.BlockSpec(memory_space=pl.ANY)] * 4,
        out_specs=pl.BlockSpec(memory_space=pl.ANY),
        out_shape=jax.ShapeDtypeStruct((BATCH, FEAT), jnp.float32),
        scratch_shapes=[
            pltpu.VMEM((BATCH, 1), jnp.float32),
            pltpu.VMEM((BATCH, 1), jnp.float32),
            pltpu.VMEM((_NBUF, _R, FEAT), jnp.float32),
            pltpu.VMEM((_NBUF, _R, FEAT), jnp.float32),
            pltpu.VMEM((_NBUF, _R, FEAT), jnp.float32),
            pltpu.SemaphoreType.DMA((2,)),
            pltpu.SemaphoreType.DMA((_NBUF, 2)),
            pltpu.SemaphoreType.DMA((_NBUF,)),
        ],
    )(coef, std, x2, e2)


def kernel(x0, timesteps, eps):
    sqrt_ac, sqrt_omac = _schedule_tables()
    coef, std = _sc_gather_fn()(timesteps.astype(jnp.int32), sqrt_ac, sqrt_omac)
    sample = _tc_scale(coef.reshape(BATCH, 1), std.reshape(BATCH, 1),
                       x0.reshape(BATCH, FEAT), eps.reshape(BATCH, FEAT))
    return (sample.reshape(x0.shape), eps)


# speedup vs baseline: 3.8529x; 2.3185x over previous
import functools

import jax
import jax.numpy as jnp
from jax import lax
from jax.experimental import pallas as pl
from jax.experimental.pallas import tpu as pltpu
from jax.experimental.pallas import tpu_sc as plsc

NUM_T = 1000
IMG_SHAPE = (3, 64, 64)
BATCH = 1024
FEAT = 3 * 64 * 64

_NC = 2
_NS = 16
_L = 16
_NW = _NC * _NS
_PER_W = BATCH // _NW
_TBL_PAD = 1024


def _schedule_tables():
    scale = 1000.0 / NUM_T
    beta = jnp.linspace(scale * 0.0001, scale * 0.02, NUM_T, dtype=jnp.float32)
    alpha_cum = jnp.cumprod(1.0 - beta, axis=0)
    sqrt_ac = jnp.sqrt(alpha_cum)
    sqrt_omac = jnp.sqrt(1.0 - alpha_cum)
    pad = _TBL_PAD - NUM_T
    return (jnp.pad(sqrt_ac, (0, pad)), jnp.pad(sqrt_omac, (0, pad)))


def _sc_gather_body(ts_hbm, ac_hbm, om_hbm, coef_hbm, std_hbm,
                    ac_v, om_v, idx_v, coef_v, std_v):
    wid = lax.axis_index("s") * _NC + lax.axis_index("c")
    base = wid * _PER_W
    pltpu.sync_copy(ac_hbm, ac_v)
    pltpu.sync_copy(om_hbm, om_v)
    pltpu.sync_copy(ts_hbm.at[pl.ds(base, _PER_W)], idx_v)
    for j in range(_PER_W // _L):
        idx = idx_v[pl.ds(j * _L, _L)]
        coef_v[pl.ds(j * _L, _L)] = plsc.load_gather(ac_v, [idx])
        std_v[pl.ds(j * _L, _L)] = plsc.load_gather(om_v, [idx])
    pltpu.sync_copy(coef_v, coef_hbm.at[pl.ds(base, _PER_W)])
    pltpu.sync_copy(std_v, std_hbm.at[pl.ds(base, _PER_W)])


@functools.lru_cache(maxsize=None)
def _sc_gather_fn():
    return pl.kernel(
        _sc_gather_body,
        out_type=(
            jax.ShapeDtypeStruct((BATCH,), jnp.float32),
            jax.ShapeDtypeStruct((BATCH,), jnp.float32),
        ),
        mesh=plsc.VectorSubcoreMesh(core_axis_name="c", subcore_axis_name="s"),
        compiler_params=pltpu.CompilerParams(needs_layout_passes=False),
        scratch_types=[
            pltpu.VMEM((_TBL_PAD,), jnp.float32),
            pltpu.VMEM((_TBL_PAD,), jnp.float32),
            pltpu.VMEM((_PER_W,), jnp.int32),
            pltpu.VMEM((_PER_W,), jnp.float32),
            pltpu.VMEM((_PER_W,), jnp.float32),
        ],
    )


_RF = 1024


def _scale_body(coef_ref, std_ref, x_ref, e_ref, out_ref):
    out_ref[...] = coef_ref[...] * x_ref[...] + std_ref[...] * e_ref[...]


def _tc_scale(coef, std, xT, eT):
    grid = (FEAT // _RF,)
    blk = pl.BlockSpec((_RF, BATCH), lambda i: (i, 0))
    row = pl.BlockSpec((1, BATCH), lambda i: (0, 0))
    return pl.pallas_call(
        _scale_body,
        grid=grid,
        in_specs=[row, row, blk, blk],
        out_specs=blk,
        out_shape=jax.ShapeDtypeStruct((FEAT, BATCH), jnp.float32),
    )(coef, std, xT, eT)


def kernel(x0, timesteps, eps):
    sqrt_ac, sqrt_omac = _schedule_tables()
    coef, std = _sc_gather_fn()(timesteps.astype(jnp.int32), sqrt_ac, sqrt_omac)
    xT = x0.transpose(1, 2, 3, 0).reshape(FEAT, BATCH)
    eT = eps.transpose(1, 2, 3, 0).reshape(FEAT, BATCH)
    outT = _tc_scale(coef.reshape(1, BATCH), std.reshape(1, BATCH), xT, eT)
    sample = outT.reshape(IMG_SHAPE + (BATCH,)).transpose(3, 0, 1, 2)
    return (sample, eps)


# speedup vs baseline: 4.5545x; 1.1821x over previous
import functools

import jax
import jax.numpy as jnp
from jax import lax
from jax.experimental import pallas as pl
from jax.experimental.pallas import tpu as pltpu
from jax.experimental.pallas import tpu_sc as plsc

NUM_T = 1000
IMG_SHAPE = (3, 64, 64)
BATCH = 1024
FEAT = 3 * 64 * 64

_NC = 2
_NS = 16
_L = 16
_NW = _NC * _NS
_PER_W = BATCH // _NW
_TBL_PAD = 1024


def _schedule_tables():
    scale = 1000.0 / NUM_T
    beta = jnp.linspace(scale * 0.0001, scale * 0.02, NUM_T, dtype=jnp.float32)
    alpha_cum = jnp.cumprod(1.0 - beta, axis=0)
    sqrt_ac = jnp.sqrt(alpha_cum)
    sqrt_omac = jnp.sqrt(1.0 - alpha_cum)
    pad = _TBL_PAD - NUM_T
    return (jnp.pad(sqrt_ac, (0, pad)), jnp.pad(sqrt_omac, (0, pad)))


def _sc_gather_body(ts_hbm, ac_hbm, om_hbm, coef_hbm, std_hbm,
                    ac_v, om_v, idx_v, coef_v, std_v):
    wid = lax.axis_index("s") * _NC + lax.axis_index("c")
    base = wid * _PER_W
    pltpu.sync_copy(ac_hbm, ac_v)
    pltpu.sync_copy(om_hbm, om_v)
    pltpu.sync_copy(ts_hbm.at[pl.ds(base, _PER_W)], idx_v)
    for j in range(_PER_W // _L):
        idx = idx_v[pl.ds(j * _L, _L)]
        coef_v[pl.ds(j * _L, _L)] = plsc.load_gather(ac_v, [idx])
        std_v[pl.ds(j * _L, _L)] = plsc.load_gather(om_v, [idx])
    pltpu.sync_copy(coef_v, coef_hbm.at[pl.ds(base, _PER_W)])
    pltpu.sync_copy(std_v, std_hbm.at[pl.ds(base, _PER_W)])


@functools.lru_cache(maxsize=None)
def _sc_gather_fn():
    return pl.kernel(
        _sc_gather_body,
        out_type=(
            jax.ShapeDtypeStruct((BATCH,), jnp.float32),
            jax.ShapeDtypeStruct((BATCH,), jnp.float32),
        ),
        mesh=plsc.VectorSubcoreMesh(core_axis_name="c", subcore_axis_name="s"),
        compiler_params=pltpu.CompilerParams(needs_layout_passes=False),
        scratch_types=[
            pltpu.VMEM((_TBL_PAD,), jnp.float32),
            pltpu.VMEM((_TBL_PAD,), jnp.float32),
            pltpu.VMEM((_PER_W,), jnp.int32),
            pltpu.VMEM((_PER_W,), jnp.float32),
            pltpu.VMEM((_PER_W,), jnp.float32),
        ],
    )


_RF = 1024


def _scale_body(coef_ref, std_ref, x_ref, e_ref, out_ref, eout_ref):
    e = e_ref[...]
    out_ref[...] = coef_ref[...] * x_ref[...] + std_ref[...] * e
    eout_ref[...] = e


def _tc_scale(coef, std, xT, eT):
    grid = (FEAT // _RF,)
    blk = pl.BlockSpec((_RF, BATCH), lambda i: (i, 0))
    row = pl.BlockSpec((1, BATCH), lambda i: (0, 0))
    return pl.pallas_call(
        _scale_body,
        grid=grid,
        in_specs=[row, row, blk, blk],
        out_specs=(blk, blk),
        out_shape=(jax.ShapeDtypeStruct((FEAT, BATCH), jnp.float32),
                   jax.ShapeDtypeStruct((FEAT, BATCH), jnp.float32)),
    )(coef, std, xT, eT)


def kernel(x0, timesteps, eps):
    sqrt_ac, sqrt_omac = _schedule_tables()
    coef, std = _sc_gather_fn()(timesteps.astype(jnp.int32), sqrt_ac, sqrt_omac)
    xT = x0.transpose(1, 2, 3, 0).reshape(FEAT, BATCH)
    eT = eps.transpose(1, 2, 3, 0).reshape(FEAT, BATCH)
    outT, eoutT = _tc_scale(coef.reshape(1, BATCH), std.reshape(1, BATCH),
                            xT, eT)
    sample = outT.reshape(IMG_SHAPE + (BATCH,)).transpose(3, 0, 1, 2)
    eps_out = eoutT.reshape(IMG_SHAPE + (BATCH,)).transpose(3, 0, 1, 2)
    return (sample, eps_out)


# speedup vs baseline: 4.6291x; 1.0164x over previous
import functools

import jax
import jax.numpy as jnp
from jax import lax
from jax.experimental import pallas as pl
from jax.experimental.pallas import tpu as pltpu
from jax.experimental.pallas import tpu_sc as plsc

NUM_T = 1000
IMG_SHAPE = (3, 64, 64)
BATCH = 1024
FEAT = 3 * 64 * 64

_NC = 2
_NS = 16
_L = 16
_NW = _NC * _NS
_PER_W = BATCH // _NW
_TBL_PAD = 1024


def _schedule_tables():
    scale = 1000.0 / NUM_T
    beta = jnp.linspace(scale * 0.0001, scale * 0.02, NUM_T, dtype=jnp.float32)
    alpha_cum = jnp.cumprod(1.0 - beta, axis=0)
    sqrt_ac = jnp.sqrt(alpha_cum)
    sqrt_omac = jnp.sqrt(1.0 - alpha_cum)
    pad = _TBL_PAD - NUM_T
    return (jnp.pad(sqrt_ac, (0, pad)), jnp.pad(sqrt_omac, (0, pad)))


def _sc_gather_body(ts_hbm, ac_hbm, om_hbm, coef_hbm, std_hbm,
                    ac_v, om_v, idx_v, coef_v, std_v):
    wid = lax.axis_index("s") * _NC + lax.axis_index("c")
    base = wid * _PER_W
    pltpu.sync_copy(ac_hbm, ac_v)
    pltpu.sync_copy(om_hbm, om_v)
    pltpu.sync_copy(ts_hbm.at[pl.ds(base, _PER_W)], idx_v)
    for j in range(_PER_W // _L):
        idx = idx_v[pl.ds(j * _L, _L)]
        coef_v[pl.ds(j * _L, _L)] = plsc.load_gather(ac_v, [idx])
        std_v[pl.ds(j * _L, _L)] = plsc.load_gather(om_v, [idx])
    pltpu.sync_copy(coef_v, coef_hbm.at[pl.ds(base, _PER_W)])
    pltpu.sync_copy(std_v, std_hbm.at[pl.ds(base, _PER_W)])


@functools.lru_cache(maxsize=None)
def _sc_gather_fn():
    return pl.kernel(
        _sc_gather_body,
        out_type=(
            jax.ShapeDtypeStruct((BATCH,), jnp.float32),
            jax.ShapeDtypeStruct((BATCH,), jnp.float32),
        ),
        mesh=plsc.VectorSubcoreMesh(core_axis_name="c", subcore_axis_name="s"),
        compiler_params=pltpu.CompilerParams(needs_layout_passes=False),
        scratch_types=[
            pltpu.VMEM((_TBL_PAD,), jnp.float32),
            pltpu.VMEM((_TBL_PAD,), jnp.float32),
            pltpu.VMEM((_PER_W,), jnp.int32),
            pltpu.VMEM((_PER_W,), jnp.float32),
            pltpu.VMEM((_PER_W,), jnp.float32),
        ],
    )


_RF = 1536


def _scale_body(coef_ref, std_ref, x_ref, e_ref, out_ref, eout_ref):
    e = e_ref[...]
    out_ref[...] = coef_ref[...] * x_ref[...] + std_ref[...] * e
    eout_ref[...] = e


def _tc_scale(coef, std, xT, eT):
    grid = (FEAT // _RF,)
    blk = pl.BlockSpec((_RF, BATCH), lambda i: (i, 0))
    row = pl.BlockSpec((1, BATCH), lambda i: (0, 0))
    return pl.pallas_call(
        _scale_body,
        grid=grid,
        in_specs=[row, row, blk, blk],
        out_specs=(blk, blk),
        out_shape=(jax.ShapeDtypeStruct((FEAT, BATCH), jnp.float32),
                   jax.ShapeDtypeStruct((FEAT, BATCH), jnp.float32)),
    )(coef, std, xT, eT)


def kernel(x0, timesteps, eps):
    sqrt_ac, sqrt_omac = _schedule_tables()
    coef, std = _sc_gather_fn()(timesteps.astype(jnp.int32), sqrt_ac, sqrt_omac)
    xT = x0.transpose(1, 2, 3, 0).reshape(FEAT, BATCH)
    eT = eps.transpose(1, 2, 3, 0).reshape(FEAT, BATCH)
    outT, eoutT = _tc_scale(coef.reshape(1, BATCH), std.reshape(1, BATCH),
                            xT, eT)
    sample = outT.reshape(IMG_SHAPE + (BATCH,)).transpose(3, 0, 1, 2)
    eps_out = eoutT.reshape(IMG_SHAPE + (BATCH,)).transpose(3, 0, 1, 2)
    return (sample, eps_out)
